# baseline (speedup 1.0000x reference)
import functools

import jax
import jax.numpy as jnp
from jax import lax
from jax.experimental import pallas as pl
from jax.experimental.pallas import tpu as pltpu
from jax.experimental.pallas import tpu_sc as plsc

_N = 10000
_E = 320000
_DIM = 16

_NW = 32
_EPT = _E // _NW
_NM = 5
_MB = _EPT // _NM
_RPT = 632
_ACC_ROWS = _RPT * 16
_DN = 1280

@functools.cache
def _sc_kernels():
    mesh = plsc.VectorSubcoreMesh(
        core_axis_name="c", subcore_axis_name="s", num_cores=2, num_subcores=16
    )

    @functools.partial(
        pl.kernel,
        out_type=jax.ShapeDtypeStruct((2, _DN, 16), jnp.float32),
        mesh=mesh,
        scratch_types=[
            pltpu.VMEM((_EPT + 128,), jnp.int32),
            pltpu.VMEM((_DN, 16), jnp.float32),
            pltpu.VMEM((_DN // 16, 16), jnp.float32),
            pltpu.VMEM_SHARED((16, _DN, 16), jnp.float32),
        ],
        compiler_params=pltpu.CompilerParams(
            needs_layout_passes=False, use_tc_tiling_on_sc=False),
    )
    def sc_degree(e_hbm, out_hbm, didx, deg, tmp, slab):
        c = lax.axis_index("c")
        s = lax.axis_index("s")
        wid = c * 16 + s
        zeros = jnp.zeros((16,), jnp.float32)
        rpt = _DN // 16

        def zbody(i, _):
            deg[i, :] = zeros
            return 0

        lax.fori_loop(0, _DN, zbody, 0)
        beg = _E + wid * _EPT
        algn = pl.multiple_of((beg // 128) * 128, 128)
        off = beg - (beg // 128) * 128
        pltpu.sync_copy(e_hbm.at[pl.ds(algn, _EPT + 128)], didx)
        ones = jnp.ones((16,), jnp.float32)

        def body(i, _):
            idx = didx[pl.ds(off + i * 16, 16)]
            plsc.addupdate_scatter(deg, [idx >> 3, idx & 7], ones)
            return 0

        lax.fori_loop(0, _EPT // 16, body, 0)
        pltpu.sync_copy(deg, slab.at[s])
        plsc.subcore_barrier()

        def zb2(i, _):
            deg[i, :] = zeros
            return 0

        lax.fori_loop(0, rpt, zb2, 0)
        for t in range(16):
            pltpu.sync_copy(slab.at[t, pl.ds(s * rpt, rpt)], tmp)

            def ab(i, _):
                deg[i, :] = deg[i, :] + tmp[i, :]
                return 0

            lax.fori_loop(0, rpt, ab, 0)
        pltpu.sync_copy(deg.at[pl.ds(0, rpt)], out_hbm.at[c, pl.ds(s * rpt, rpt)])

    @functools.partial(
        pl.kernel,
        out_type=jax.ShapeDtypeStruct((2, _ACC_ROWS, _DIM), jnp.float32),
        mesh=mesh,
        scratch_types=[
            pltpu.VMEM((_EPT + 128,), jnp.int32),
            pltpu.VMEM((_EPT + 128,), jnp.int32),
            [pltpu.VMEM((_MB, _DIM), jnp.float32)] * 2,
            pltpu.VMEM((_RPT, _DIM), jnp.float32),
            pltpu.VMEM_SHARED((_ACC_ROWS, _DIM), jnp.float32),
            pltpu.VMEM_SHARED((_ACC_ROWS, _DIM), jnp.float32),
            [pltpu.SemaphoreType.DMA] * 4,
        ],
        compiler_params=pltpu.CompilerParams(use_tc_tiling_on_sc=False),
    )
    def sc_agg(tab_hbm, e_hbm, out_hbm, sidx, didx, rows, buf, acc, tabs, sems):
        c = lax.axis_index("c")
        s = lax.axis_index("s")
        wid = c * 16 + s
        gsem = [sems[0], sems[1]]
        ssem = [sems[2], sems[3]]
        zeros = jnp.zeros((16,), jnp.float32)

        def zbody(i, _):
            buf[i, :] = zeros
            return 0

        pltpu.sync_copy(tab_hbm.at[pl.ds(s * _RPT, _RPT)], buf)
        pltpu.sync_copy(buf, tabs.at[pl.ds(s * _RPT, _RPT)])
        lax.fori_loop(0, _RPT, zbody, 0)
        pltpu.sync_copy(buf, acc.at[pl.ds(s * _RPT, _RPT)])

        sbeg = wid * _EPT
        dbeg = _E + wid * _EPT
        soff = sbeg - (sbeg // 128) * 128
        doff = dbeg - (dbeg // 128) * 128
        pltpu.sync_copy(
            e_hbm.at[pl.ds(pl.multiple_of((sbeg // 128) * 128, 128), _EPT + 128)], sidx)
        pltpu.sync_copy(
            e_hbm.at[pl.ds(pl.multiple_of((dbeg // 128) * 128, 128), _EPT + 128)], didx)
        plsc.subcore_barrier()

        def gat(m, b):
            return pltpu.make_async_copy(
                tabs.at[sidx.at[pl.ds(soff + m * _MB, _MB)]], rows[b], gsem[b])

        def sca(m, b):
            return pltpu.make_async_copy(
                rows[b], acc.at[didx.at[pl.ds(doff + m * _MB, _MB)]], ssem[b])

        gat(0, 0).start()
        for m in range(_NM):
            b = m % 2
            gat(m, b).wait()
            if m + 1 < _NM:
                if m >= 1:
                    sca(m - 1, 1 - b).wait()
                gat(m + 1, 1 - b).start()
            pltpu.async_copy(rows[b], acc.at[didx.at[pl.ds(doff + m * _MB, _MB)]],
                             ssem[b], add=True)
        sca(_NM - 2, (_NM - 2) % 2).wait()
        sca(_NM - 1, (_NM - 1) % 2).wait()
        plsc.subcore_barrier()
        pltpu.sync_copy(acc.at[pl.ds(s * _RPT, _RPT)], buf)
        pltpu.sync_copy(buf, out_hbm.at[c, pl.ds(s * _RPT, _RPT)])

    return sc_degree, sc_agg


_VR = _ACC_ROWS * _DIM // 128


def _tc1_body(degp_ref, x_ref, w1_ref, dis16_ref, hs_ref, hself_ref):
    degp = degp_ref[0] + degp_ref[1]
    disp = lax.rsqrt(degp + 1.0)[:, 0:8]
    r8 = lax.broadcasted_iota(jnp.int32, (8, 128), 0)
    c8 = lax.broadcasted_iota(jnp.int32, (8, 128), 1) // _DIM
    expand = jnp.where(r8 == c8, 1.0, 0.0)
    dis16 = jnp.dot(disp, expand, preferred_element_type=jnp.float32)[0:_VR, :]
    h = jnp.dot(x_ref[...], w1_ref[...], preferred_element_type=jnp.float32)
    hp = jnp.concatenate([h, jnp.zeros((_ACC_ROWS - _N, _DIM), jnp.float32)], 0)
    h3 = hp.reshape(_VR, 8, _DIM)
    kk = lax.broadcasted_iota(jnp.int32, (_DIM, 128), 0)
    cc = lax.broadcasted_iota(jnp.int32, (_DIM, 128), 1)
    hv = jnp.zeros((_VR, 128), jnp.float32)
    for j in range(8):
        ej = jnp.where(cc == kk + _DIM * j, 1.0, 0.0)
        hv = hv + jnp.dot(h3[:, j, :], ej, preferred_element_type=jnp.float32)
    hs = hv * dis16
    dis16_ref[...] = dis16
    hs_ref[...] = hs
    hself_ref[...] = hs * dis16


_tc1 = pl.pallas_call(
    _tc1_body,
    out_shape=(
        jax.ShapeDtypeStruct((_VR, 128), jnp.float32),
        jax.ShapeDtypeStruct((_VR, 128), jnp.float32),
        jax.ShapeDtypeStruct((_VR, 128), jnp.float32),
    ),
)


def _tile8(mat):
    r = lax.broadcasted_iota(jnp.int32, (128, 128), 0)
    c = lax.broadcasted_iota(jnp.int32, (128, 128), 1)
    tiled = jnp.tile(mat, (8, 8))
    return jnp.where(r // _DIM == c // _DIM, tiled, 0.0)


def _tc2_body(acc_ref, dis16_ref, hself_ref, b1_ref, w2_ref, gs_ref, gself_ref):
    av = acc_ref[0:_VR, :] + acc_ref[_VR:2 * _VR, :]
    dis16 = dis16_ref[...]
    b1v = jnp.tile(b1_ref[...], (8,))
    z = av * dis16 + hself_ref[...] + b1v[None, :]
    h2 = jnp.maximum(z, 0.0)
    g = jnp.dot(h2, _tile8(w2_ref[...]), preferred_element_type=jnp.float32)
    gs = g * dis16
    gs_ref[...] = gs
    gself_ref[...] = gs * dis16


_tc2 = pl.pallas_call(
    _tc2_body,
    out_shape=(
        jax.ShapeDtypeStruct((_VR, 128), jnp.float32),
        jax.ShapeDtypeStruct((_VR, 128), jnp.float32),
    ),
)


def _tc3_body(acc_ref, dis16_ref, gself_ref, b2_ref, out_ref):
    av = acc_ref[0:_VR, :] + acc_ref[_VR:2 * _VR, :]
    b2v = jnp.tile(b2_ref[...], (8,))
    logitsv = av * dis16_ref[...] + gself_ref[...] + b2v[None, :]
    m = jnp.concatenate(
        [jnp.broadcast_to(
            jnp.max(logitsv[:, _DIM * j:_DIM * (j + 1)], axis=1, keepdims=True),
            (_VR, _DIM)) for j in range(8)], axis=1)
    ex = jnp.exp(logitsv - m)
    lse = jnp.concatenate(
        [jnp.broadcast_to(
            jnp.log(jnp.sum(ex[:, _DIM * j:_DIM * (j + 1)], axis=1, keepdims=True)),
            (_VR, _DIM)) for j in range(8)], axis=1) + m
    out_ref[...] = (logitsv - lse)[0:_N * _DIM // 128, :]


_tc3 = pl.pallas_call(
    _tc3_body,
    out_shape=jax.ShapeDtypeStruct((_N * _DIM // 128, 128), jnp.float32),
)


def kernel(x, edge_index, W1, b1, W2, b2):
    ei = edge_index.astype(jnp.int32).reshape(-1)

    sc_degree, sc_agg = _sc_kernels()
    degp = sc_degree(ei)
    dis16, hsv, hselfv = _tc1(degp, x, W1)
    acc1 = sc_agg(hsv.reshape(_ACC_ROWS, _DIM), ei).reshape(2 * _VR, 128)
    gsv, gselfv = _tc2(acc1, dis16, hselfv, b1, W2)
    acc2 = sc_agg(gsv.reshape(_ACC_ROWS, _DIM), ei).reshape(2 * _VR, 128)
    predv = _tc3(acc2, dis16, gselfv, b2)
    return predv.reshape(_N, _DIM)


# speedup vs baseline: 1.0758x; 1.0758x over previous
import functools

import jax
import jax.numpy as jnp
from jax import lax
from jax.experimental import pallas as pl
from jax.experimental.pallas import tpu as pltpu
from jax.experimental.pallas import tpu_sc as plsc

_N = 10000
_E = 320000
_DIM = 16

_NW = 32
_EPT = _E // _NW
_NM = 5
_MB = _EPT // _NM
_RPT = 632
_ACC_ROWS = _RPT * 16
_DN = 1280

@functools.cache
def _sc_kernels():
    mesh = plsc.VectorSubcoreMesh(
        core_axis_name="c", subcore_axis_name="s", num_cores=2, num_subcores=16
    )

    @functools.partial(
        pl.kernel,
        out_type=jax.ShapeDtypeStruct((2, _DN, 16), jnp.float32),
        mesh=mesh,
        scratch_types=[
            pltpu.VMEM((_EPT + 128,), jnp.int32),
            pltpu.VMEM((_DN, 16), jnp.float32),
            pltpu.VMEM((_DN,), jnp.int32),
            pltpu.VMEM_SHARED((_DN, 16), jnp.float32),
        ],
        compiler_params=pltpu.CompilerParams(
            needs_layout_passes=False, use_tc_tiling_on_sc=False),
    )
    def sc_degree(e_hbm, out_hbm, didx, deg, idr, deg_s):
        c = lax.axis_index("c")
        s = lax.axis_index("s")
        wid = c * 16 + s
        zeros = jnp.zeros((16,), jnp.float32)

        def zbody(i, _):
            deg[i, :] = zeros
            return 0

        lax.fori_loop(0, _DN, zbody, 0)
        pltpu.sync_copy(deg.at[pl.ds(0, _DN // 16)], deg_s.at[pl.ds(s * (_DN // 16), _DN // 16)])
        beg = _E + wid * _EPT
        algn = pl.multiple_of((beg // 128) * 128, 128)
        off = beg - (beg // 128) * 128
        pltpu.sync_copy(e_hbm.at[pl.ds(algn, _EPT + 128)], didx)
        ones = jnp.ones((16,), jnp.float32)

        def body(i, _):
            idx = didx[pl.ds(off + i * 16, 16)]
            plsc.addupdate_scatter(deg, [idx >> 3, idx & 7], ones)
            return 0

        lax.fori_loop(0, _EPT // 16, body, 0)
        iota = lax.iota(jnp.int32, 16)

        def ibody(i, _):
            idr[pl.ds(i * 16, 16)] = iota + i * 16
            return 0

        lax.fori_loop(0, _DN // 16, ibody, 0)
        plsc.subcore_barrier()
        pltpu.sync_copy(deg, deg_s.at[idr], add=True)
        plsc.subcore_barrier()
        pltpu.sync_copy(deg_s.at[pl.ds(s * (_DN // 16), _DN // 16)], deg.at[pl.ds(0, _DN // 16)])
        pltpu.sync_copy(deg.at[pl.ds(0, _DN // 16)], out_hbm.at[c, pl.ds(s * (_DN // 16), _DN // 16)])

    @functools.partial(
        pl.kernel,
        out_type=jax.ShapeDtypeStruct((2, _ACC_ROWS, _DIM), jnp.float32),
        mesh=mesh,
        scratch_types=[
            pltpu.VMEM((_EPT + 128,), jnp.int32),
            pltpu.VMEM((_EPT + 128,), jnp.int32),
            [pltpu.VMEM((_MB, _DIM), jnp.float32)] * 2,
            pltpu.VMEM((_RPT, _DIM), jnp.float32),
            pltpu.VMEM_SHARED((_ACC_ROWS, _DIM), jnp.float32),
            pltpu.VMEM_SHARED((_ACC_ROWS, _DIM), jnp.float32),
            [pltpu.SemaphoreType.DMA] * 4,
        ],
        compiler_params=pltpu.CompilerParams(use_tc_tiling_on_sc=False),
    )
    def sc_agg(tab_hbm, e_hbm, out_hbm, sidx, didx, rows, buf, acc, tabs, sems):
        c = lax.axis_index("c")
        s = lax.axis_index("s")
        wid = c * 16 + s
        gsem = [sems[0], sems[1]]
        ssem = [sems[2], sems[3]]
        zeros = jnp.zeros((16,), jnp.float32)

        def zbody(i, _):
            buf[i, :] = zeros
            return 0

        pltpu.sync_copy(tab_hbm.at[pl.ds(s * _RPT, _RPT)], buf)
        pltpu.sync_copy(buf, tabs.at[pl.ds(s * _RPT, _RPT)])
        lax.fori_loop(0, _RPT, zbody, 0)
        pltpu.sync_copy(buf, acc.at[pl.ds(s * _RPT, _RPT)])

        sbeg = wid * _EPT
        dbeg = _E + wid * _EPT
        soff = sbeg - (sbeg // 128) * 128
        doff = dbeg - (dbeg // 128) * 128
        pltpu.sync_copy(
            e_hbm.at[pl.ds(pl.multiple_of((sbeg // 128) * 128, 128), _EPT + 128)], sidx)
        pltpu.sync_copy(
            e_hbm.at[pl.ds(pl.multiple_of((dbeg // 128) * 128, 128), _EPT + 128)], didx)
        plsc.subcore_barrier()

        def gat(m, b):
            return pltpu.make_async_copy(
                tabs.at[sidx.at[pl.ds(soff + m * _MB, _MB)]], rows[b], gsem[b])

        def sca(m, b):
            return pltpu.make_async_copy(
                rows[b], acc.at[didx.at[pl.ds(doff + m * _MB, _MB)]], ssem[b])

        gat(0, 0).start()
        for m in range(_NM):
            b = m % 2
            gat(m, b).wait()
            if m + 1 < _NM:
                if m >= 1:
                    sca(m - 1, 1 - b).wait()
                gat(m + 1, 1 - b).start()
            pltpu.async_copy(rows[b], acc.at[didx.at[pl.ds(doff + m * _MB, _MB)]],
                             ssem[b], add=True)
        sca(_NM - 2, (_NM - 2) % 2).wait()
        sca(_NM - 1, (_NM - 1) % 2).wait()
        plsc.subcore_barrier()
        pltpu.sync_copy(acc.at[pl.ds(s * _RPT, _RPT)], buf)
        pltpu.sync_copy(buf, out_hbm.at[c, pl.ds(s * _RPT, _RPT)])

    return sc_degree, sc_agg


_VR = _ACC_ROWS * _DIM // 128


def _tc1_body(degp_ref, x_ref, w1_ref, dis16_ref, hs_ref, hself_ref):
    degp = degp_ref[0] + degp_ref[1]
    disp = lax.rsqrt(degp + 1.0)[:, 0:8]
    r8 = lax.broadcasted_iota(jnp.int32, (8, 128), 0)
    c8 = lax.broadcasted_iota(jnp.int32, (8, 128), 1) // _DIM
    expand = jnp.where(r8 == c8, 1.0, 0.0)
    dis16 = jnp.dot(disp, expand, preferred_element_type=jnp.float32)[0:_VR, :]
    h = jnp.dot(x_ref[...], w1_ref[...], preferred_element_type=jnp.float32)
    hp = jnp.concatenate([h, jnp.zeros((_ACC_ROWS - _N, _DIM), jnp.float32)], 0)
    h3 = hp.reshape(_VR, 8, _DIM)
    kk = lax.broadcasted_iota(jnp.int32, (_DIM, 128), 0)
    cc = lax.broadcasted_iota(jnp.int32, (_DIM, 128), 1)
    hv = jnp.zeros((_VR, 128), jnp.float32)
    for j in range(8):
        ej = jnp.where(cc == kk + _DIM * j, 1.0, 0.0)
        hv = hv + jnp.dot(h3[:, j, :], ej, preferred_element_type=jnp.float32)
    hs = hv * dis16
    dis16_ref[...] = dis16
    hs_ref[...] = hs
    hself_ref[...] = hs * dis16


_tc1 = pl.pallas_call(
    _tc1_body,
    out_shape=(
        jax.ShapeDtypeStruct((_VR, 128), jnp.float32),
        jax.ShapeDtypeStruct((_VR, 128), jnp.float32),
        jax.ShapeDtypeStruct((_VR, 128), jnp.float32),
    ),
)


def _tile8(mat):
    r = lax.broadcasted_iota(jnp.int32, (128, 128), 0)
    c = lax.broadcasted_iota(jnp.int32, (128, 128), 1)
    tiled = jnp.tile(mat, (8, 8))
    return jnp.where(r // _DIM == c // _DIM, tiled, 0.0)


def _tc2_body(acc_ref, dis16_ref, hself_ref, b1_ref, w2_ref, gs_ref, gself_ref):
    av = acc_ref[0:_VR, :] + acc_ref[_VR:2 * _VR, :]
    dis16 = dis16_ref[...]
    b1v = jnp.tile(b1_ref[...], (8,))
    z = av * dis16 + hself_ref[...] + b1v[None, :]
    h2 = jnp.maximum(z, 0.0)
    g = jnp.dot(h2, _tile8(w2_ref[...]), preferred_element_type=jnp.float32)
    gs = g * dis16
    gs_ref[...] = gs
    gself_ref[...] = gs * dis16


_tc2 = pl.pallas_call(
    _tc2_body,
    out_shape=(
        jax.ShapeDtypeStruct((_VR, 128), jnp.float32),
        jax.ShapeDtypeStruct((_VR, 128), jnp.float32),
    ),
)


def _tc3_body(acc_ref, dis16_ref, gself_ref, b2_ref, out_ref):
    av = acc_ref[0:_VR, :] + acc_ref[_VR:2 * _VR, :]
    b2v = jnp.tile(b2_ref[...], (8,))
    logitsv = av * dis16_ref[...] + gself_ref[...] + b2v[None, :]
    m = jnp.concatenate(
        [jnp.broadcast_to(
            jnp.max(logitsv[:, _DIM * j:_DIM * (j + 1)], axis=1, keepdims=True),
            (_VR, _DIM)) for j in range(8)], axis=1)
    ex = jnp.exp(logitsv - m)
    lse = jnp.concatenate(
        [jnp.broadcast_to(
            jnp.log(jnp.sum(ex[:, _DIM * j:_DIM * (j + 1)], axis=1, keepdims=True)),
            (_VR, _DIM)) for j in range(8)], axis=1) + m
    out_ref[...] = (logitsv - lse)[0:_N * _DIM // 128, :]


_tc3 = pl.pallas_call(
    _tc3_body,
    out_shape=jax.ShapeDtypeStruct((_N * _DIM // 128, 128), jnp.float32),
)


def kernel(x, edge_index, W1, b1, W2, b2):
    ei = edge_index.astype(jnp.int32).reshape(-1)

    sc_degree, sc_agg = _sc_kernels()
    degp = sc_degree(ei)
    dis16, hsv, hselfv = _tc1(degp, x, W1)
    acc1 = sc_agg(hsv.reshape(_ACC_ROWS, _DIM), ei).reshape(2 * _VR, 128)
    gsv, gselfv = _tc2(acc1, dis16, hselfv, b1, W2)
    acc2 = sc_agg(gsv.reshape(_ACC_ROWS, _DIM), ei).reshape(2 * _VR, 128)
    predv = _tc3(acc2, dis16, gselfv, b2)
    return predv.reshape(_N, _DIM)


# speedup vs baseline: 1.1119x; 1.0336x over previous
import functools

import jax
import jax.numpy as jnp
from jax import lax
from jax.experimental import pallas as pl
from jax.experimental.pallas import tpu as pltpu
from jax.experimental.pallas import tpu_sc as plsc

_N = 10000
_E = 320000
_DIM = 16

_NW = 32
_EPT = _E // _NW
_NM = 5
_MB = _EPT // _NM
_RPT = 632
_ACC_ROWS = _RPT * 16
_DN = 1280

@functools.cache
def _sc_kernels():
    mesh = plsc.VectorSubcoreMesh(
        core_axis_name="c", subcore_axis_name="s", num_cores=2, num_subcores=16
    )

    @functools.partial(
        pl.kernel,
        out_type=jax.ShapeDtypeStruct((2, _DN, 16), jnp.float32),
        mesh=mesh,
        scratch_types=[
            pltpu.VMEM((_EPT + 128,), jnp.int32),
            pltpu.VMEM((_DN, 16), jnp.float32),
            pltpu.VMEM((_DN,), jnp.int32),
            pltpu.VMEM_SHARED((_DN, 16), jnp.float32),
        ],
        compiler_params=pltpu.CompilerParams(
            needs_layout_passes=False, use_tc_tiling_on_sc=False),
    )
    def sc_degree(e_hbm, out_hbm, didx, deg, idr, deg_s):
        c = lax.axis_index("c")
        s = lax.axis_index("s")
        wid = c * 16 + s
        zeros = jnp.zeros((16,), jnp.float32)

        def zbody(i, _):
            deg[i, :] = zeros
            return 0

        lax.fori_loop(0, _DN, zbody, 0)
        pltpu.sync_copy(deg.at[pl.ds(0, _DN // 16)], deg_s.at[pl.ds(s * (_DN // 16), _DN // 16)])
        beg = _E + wid * _EPT
        algn = pl.multiple_of((beg // 128) * 128, 128)
        off = beg - (beg // 128) * 128
        pltpu.sync_copy(e_hbm.at[pl.ds(algn, _EPT + 128)], didx)
        ones = jnp.ones((16,), jnp.float32)

        def body(i, _):
            idx = didx[pl.ds(off + i * 16, 16)]
            plsc.addupdate_scatter(deg, [idx >> 3, idx & 7], ones)
            return 0

        lax.fori_loop(0, _EPT // 16, body, 0)
        iota = lax.iota(jnp.int32, 16)

        def ibody(i, _):
            idr[pl.ds(i * 16, 16)] = iota + i * 16
            return 0

        lax.fori_loop(0, _DN // 16, ibody, 0)
        plsc.subcore_barrier()
        pltpu.sync_copy(deg, deg_s.at[idr], add=True)
        plsc.subcore_barrier()
        pltpu.sync_copy(deg_s.at[pl.ds(s * (_DN // 16), _DN // 16)], deg.at[pl.ds(0, _DN // 16)])
        pltpu.sync_copy(deg.at[pl.ds(0, _DN // 16)], out_hbm.at[c, pl.ds(s * (_DN // 16), _DN // 16)])

    @functools.partial(
        pl.kernel,
        out_type=jax.ShapeDtypeStruct((2, _ACC_ROWS, _DIM), jnp.float32),
        mesh=mesh,
        scratch_types=[
            pltpu.VMEM((_EPT + 128,), jnp.int32),
            pltpu.VMEM((_EPT + 128,), jnp.int32),
            [pltpu.VMEM((_MB, _DIM), jnp.float32)] * 2,
            pltpu.VMEM((_RPT, _DIM), jnp.float32),
            pltpu.VMEM_SHARED((_ACC_ROWS, _DIM), jnp.float32),
            pltpu.VMEM_SHARED((_ACC_ROWS, _DIM), jnp.float32),
            [pltpu.SemaphoreType.DMA] * 4,
        ],
        compiler_params=pltpu.CompilerParams(use_tc_tiling_on_sc=False),
    )
    def sc_agg(tab_hbm, e_hbm, out_hbm, sidx, didx, rows, buf, acc, tabs, sems):
        c = lax.axis_index("c")
        s = lax.axis_index("s")
        wid = c * 16 + s
        gsem = [sems[0], sems[1]]
        ssem = [sems[2], sems[3]]
        zeros = jnp.zeros((16,), jnp.float32)

        def zbody(i, _):
            buf[i, :] = zeros
            return 0

        pltpu.sync_copy(tab_hbm.at[pl.ds(s * _RPT, _RPT)], buf)
        pltpu.sync_copy(buf, tabs.at[pl.ds(s * _RPT, _RPT)])
        lax.fori_loop(0, _RPT, zbody, 0)
        pltpu.sync_copy(buf, acc.at[pl.ds(s * _RPT, _RPT)])

        sbeg = wid * _EPT
        dbeg = _E + wid * _EPT
        soff = sbeg - (sbeg // 128) * 128
        doff = dbeg - (dbeg // 128) * 128
        pltpu.sync_copy(
            e_hbm.at[pl.ds(pl.multiple_of((sbeg // 128) * 128, 128), _EPT + 128)], sidx)
        pltpu.sync_copy(
            e_hbm.at[pl.ds(pl.multiple_of((dbeg // 128) * 128, 128), _EPT + 128)], didx)
        plsc.subcore_barrier()

        def gat(m, b):
            return pltpu.make_async_copy(
                tabs.at[sidx.at[pl.ds(soff + m * _MB, _MB)]], rows[b], gsem[b])

        def sca(m, b):
            return pltpu.make_async_copy(
                rows[b], acc.at[didx.at[pl.ds(doff + m * _MB, _MB)]], ssem[b])

        gat(0, 0).start()
        for m in range(_NM):
            b = m % 2
            gat(m, b).wait()
            if m + 1 < _NM:
                if m >= 1:
                    sca(m - 1, 1 - b).wait()
                gat(m + 1, 1 - b).start()
            pltpu.async_copy(rows[b], acc.at[didx.at[pl.ds(doff + m * _MB, _MB)]],
                             ssem[b], add=True)
        sca(_NM - 2, (_NM - 2) % 2).wait()
        sca(_NM - 1, (_NM - 1) % 2).wait()
        plsc.subcore_barrier()
        pltpu.sync_copy(acc.at[pl.ds(s * _RPT, _RPT)], buf)
        pltpu.sync_copy(buf, out_hbm.at[c, pl.ds(s * _RPT, _RPT)])

    return sc_degree, sc_agg


_VR = _ACC_ROWS * _DIM // 128


def _tc1a_body(x_ref, w1_ref, hv_ref):
    h = jnp.dot(x_ref[...], w1_ref[...], preferred_element_type=jnp.float32)
    hp = jnp.concatenate([h, jnp.zeros((_ACC_ROWS - _N, _DIM), jnp.float32)], 0)
    h3 = hp.reshape(_VR, 8, _DIM)
    kk = lax.broadcasted_iota(jnp.int32, (_DIM, 128), 0)
    cc = lax.broadcasted_iota(jnp.int32, (_DIM, 128), 1)
    hv = jnp.zeros((_VR, 128), jnp.float32)
    for j in range(8):
        ej = jnp.where(cc == kk + _DIM * j, 1.0, 0.0)
        hv = hv + jnp.dot(h3[:, j, :], ej, preferred_element_type=jnp.float32)
    hv_ref[...] = hv


_tc1a = pl.pallas_call(
    _tc1a_body,
    out_shape=jax.ShapeDtypeStruct((_VR, 128), jnp.float32),
)


def _tc1b_body(degp_ref, hv_ref, dis16_ref, hs_ref, hself_ref):
    degp = degp_ref[0] + degp_ref[1]
    disp = lax.rsqrt(degp + 1.0)[:, 0:8]
    r8 = lax.broadcasted_iota(jnp.int32, (8, 128), 0)
    c8 = lax.broadcasted_iota(jnp.int32, (8, 128), 1) // _DIM
    expand = jnp.where(r8 == c8, 1.0, 0.0)
    dis16 = jnp.dot(disp, expand, preferred_element_type=jnp.float32)[0:_VR, :]
    hs = hv_ref[...] * dis16
    dis16_ref[...] = dis16
    hs_ref[...] = hs
    hself_ref[...] = hs * dis16


_tc1b = pl.pallas_call(
    _tc1b_body,
    out_shape=(
        jax.ShapeDtypeStruct((_VR, 128), jnp.float32),
        jax.ShapeDtypeStruct((_VR, 128), jnp.float32),
        jax.ShapeDtypeStruct((_VR, 128), jnp.float32),
    ),
)


def _tile8(mat):
    r = lax.broadcasted_iota(jnp.int32, (128, 128), 0)
    c = lax.broadcasted_iota(jnp.int32, (128, 128), 1)
    tiled = jnp.tile(mat, (8, 8))
    return jnp.where(r // _DIM == c // _DIM, tiled, 0.0)


def _tc2_body(acc_ref, dis16_ref, hself_ref, b1_ref, w2_ref, gs_ref, gself_ref):
    av = acc_ref[0:_VR, :] + acc_ref[_VR:2 * _VR, :]
    dis16 = dis16_ref[...]
    b1v = jnp.tile(b1_ref[...], (8,))
    z = av * dis16 + hself_ref[...] + b1v[None, :]
    h2 = jnp.maximum(z, 0.0)
    g = jnp.dot(h2, _tile8(w2_ref[...]), preferred_element_type=jnp.float32)
    gs = g * dis16
    gs_ref[...] = gs
    gself_ref[...] = gs * dis16


_tc2 = pl.pallas_call(
    _tc2_body,
    out_shape=(
        jax.ShapeDtypeStruct((_VR, 128), jnp.float32),
        jax.ShapeDtypeStruct((_VR, 128), jnp.float32),
    ),
)


def _tc3_body(acc_ref, dis16_ref, gself_ref, b2_ref, out_ref):
    av = acc_ref[0:_VR, :] + acc_ref[_VR:2 * _VR, :]
    b2v = jnp.tile(b2_ref[...], (8,))
    logitsv = av * dis16_ref[...] + gself_ref[...] + b2v[None, :]
    m = jnp.concatenate(
        [jnp.broadcast_to(
            jnp.max(logitsv[:, _DIM * j:_DIM * (j + 1)], axis=1, keepdims=True),
            (_VR, _DIM)) for j in range(8)], axis=1)
    ex = jnp.exp(logitsv - m)
    lse = jnp.concatenate(
        [jnp.broadcast_to(
            jnp.log(jnp.sum(ex[:, _DIM * j:_DIM * (j + 1)], axis=1, keepdims=True)),
            (_VR, _DIM)) for j in range(8)], axis=1) + m
    out_ref[...] = (logitsv - lse)[0:_N * _DIM // 128, :]


_tc3 = pl.pallas_call(
    _tc3_body,
    out_shape=jax.ShapeDtypeStruct((_N * _DIM // 128, 128), jnp.float32),
)


def kernel(x, edge_index, W1, b1, W2, b2):
    ei = edge_index.astype(jnp.int32).reshape(-1)

    sc_degree, sc_agg = _sc_kernels()
    hvr = _tc1a(x, W1)
    degp = sc_degree(ei)
    dis16, hsv, hselfv = _tc1b(degp, hvr)
    acc1 = sc_agg(hsv.reshape(_ACC_ROWS, _DIM), ei).reshape(2 * _VR, 128)
    gsv, gselfv = _tc2(acc1, dis16, hselfv, b1, W2)
    acc2 = sc_agg(gsv.reshape(_ACC_ROWS, _DIM), ei).reshape(2 * _VR, 128)
    predv = _tc3(acc2, dis16, gselfv, b2)
    return predv.reshape(_N, _DIM)


# speedup vs baseline: 1.1665x; 1.0491x over previous
import functools

import jax
import jax.numpy as jnp
from jax import lax
from jax.experimental import pallas as pl
from jax.experimental.pallas import tpu as pltpu
from jax.experimental.pallas import tpu_sc as plsc

_N = 10000
_E = 320000
_DIM = 16

_NW = 32
_EPT = _E // _NW
_NM = 5
_MB = _EPT // _NM
_RPT = 632
_ACC_ROWS = _RPT * 16
_DN = 1280

@functools.cache
def _sc_kernels():
    mesh = plsc.VectorSubcoreMesh(
        core_axis_name="c", subcore_axis_name="s", num_cores=2, num_subcores=16
    )

    @functools.partial(
        pl.kernel,
        out_type=jax.ShapeDtypeStruct((2, _DN, 16), jnp.float32),
        mesh=mesh,
        scratch_types=[
            pltpu.VMEM((_EPT + 128,), jnp.int32),
            pltpu.VMEM((_DN, 16), jnp.float32),
            pltpu.VMEM((_DN,), jnp.int32),
            pltpu.VMEM_SHARED((_DN, 16), jnp.float32),
        ],
        compiler_params=pltpu.CompilerParams(
            needs_layout_passes=False, use_tc_tiling_on_sc=False),
    )
    def sc_degree(e_hbm, out_hbm, didx, deg, idr, deg_s):
        c = lax.axis_index("c")
        s = lax.axis_index("s")
        wid = c * 16 + s
        zeros = jnp.zeros((16,), jnp.float32)

        def zbody(i, _):
            for u in range(8):
                deg[i * 8 + u, :] = zeros
            return 0

        lax.fori_loop(0, _DN // 8, zbody, 0)
        pltpu.sync_copy(deg.at[pl.ds(0, _DN // 16)], deg_s.at[pl.ds(s * (_DN // 16), _DN // 16)])
        beg = _E + wid * _EPT
        algn = pl.multiple_of((beg // 128) * 128, 128)
        off = beg - (beg // 128) * 128
        pltpu.sync_copy(e_hbm.at[pl.ds(algn, _EPT + 128)], didx)
        ones = jnp.ones((16,), jnp.float32)

        def body(i, _):
            for u in range(5):
                idx = didx[pl.ds(off + (i * 5 + u) * 16, 16)]
                plsc.addupdate_scatter(deg, [idx >> 3, idx & 7], ones)
            return 0

        lax.fori_loop(0, _EPT // 80, body, 0)
        iota = lax.iota(jnp.int32, 16)

        def ibody(i, _):
            idr[pl.ds(i * 16, 16)] = iota + i * 16
            return 0

        lax.fori_loop(0, _DN // 16, ibody, 0)
        plsc.subcore_barrier()
        pltpu.sync_copy(deg, deg_s.at[idr], add=True)
        plsc.subcore_barrier()
        pltpu.sync_copy(deg_s.at[pl.ds(s * (_DN // 16), _DN // 16)], deg.at[pl.ds(0, _DN // 16)])
        pltpu.sync_copy(deg.at[pl.ds(0, _DN // 16)], out_hbm.at[c, pl.ds(s * (_DN // 16), _DN // 16)])

    @functools.partial(
        pl.kernel,
        out_type=jax.ShapeDtypeStruct((2, _ACC_ROWS, _DIM), jnp.float32),
        mesh=mesh,
        scratch_types=[
            pltpu.VMEM((_EPT + 128,), jnp.int32),
            pltpu.VMEM((_EPT + 128,), jnp.int32),
            [pltpu.VMEM((_MB, _DIM), jnp.float32)] * 2,
            pltpu.VMEM((_RPT, _DIM), jnp.float32),
            pltpu.VMEM_SHARED((_ACC_ROWS, _DIM), jnp.float32),
            pltpu.VMEM_SHARED((_ACC_ROWS, _DIM), jnp.float32),
            [pltpu.SemaphoreType.DMA] * 4,
        ],
        compiler_params=pltpu.CompilerParams(use_tc_tiling_on_sc=False),
    )
    def sc_agg(tab_hbm, e_hbm, out_hbm, sidx, didx, rows, buf, acc, tabs, sems):
        c = lax.axis_index("c")
        s = lax.axis_index("s")
        wid = c * 16 + s
        gsem = [sems[0], sems[1]]
        ssem = [sems[2], sems[3]]
        zeros = jnp.zeros((16,), jnp.float32)

        def zbody(i, _):
            buf[i, :] = zeros
            return 0

        pltpu.sync_copy(tab_hbm.at[pl.ds(s * _RPT, _RPT)], buf)
        pltpu.sync_copy(buf, tabs.at[pl.ds(s * _RPT, _RPT)])
        lax.fori_loop(0, _RPT, zbody, 0)
        pltpu.sync_copy(buf, acc.at[pl.ds(s * _RPT, _RPT)])

        sbeg = wid * _EPT
        dbeg = _E + wid * _EPT
        soff = sbeg - (sbeg // 128) * 128
        doff = dbeg - (dbeg // 128) * 128
        pltpu.sync_copy(
            e_hbm.at[pl.ds(pl.multiple_of((sbeg // 128) * 128, 128), _EPT + 128)], sidx)
        pltpu.sync_copy(
            e_hbm.at[pl.ds(pl.multiple_of((dbeg // 128) * 128, 128), _EPT + 128)], didx)
        plsc.subcore_barrier()

        def gat(m, b):
            return pltpu.make_async_copy(
                tabs.at[sidx.at[pl.ds(soff + m * _MB, _MB)]], rows[b], gsem[b])

        def sca(m, b):
            return pltpu.make_async_copy(
                rows[b], acc.at[didx.at[pl.ds(doff + m * _MB, _MB)]], ssem[b])

        gat(0, 0).start()
        for m in range(_NM):
            b = m % 2
            gat(m, b).wait()
            if m + 1 < _NM:
                if m >= 1:
                    sca(m - 1, 1 - b).wait()
                gat(m + 1, 1 - b).start()
            pltpu.async_copy(rows[b], acc.at[didx.at[pl.ds(doff + m * _MB, _MB)]],
                             ssem[b], add=True)
        sca(_NM - 2, (_NM - 2) % 2).wait()
        sca(_NM - 1, (_NM - 1) % 2).wait()
        plsc.subcore_barrier()
        pltpu.sync_copy(acc.at[pl.ds(s * _RPT, _RPT)], buf)
        pltpu.sync_copy(buf, out_hbm.at[c, pl.ds(s * _RPT, _RPT)])

    return sc_degree, sc_agg


_VR = _ACC_ROWS * _DIM // 128


def _tc1a_body(x_ref, w1_ref, hv_ref):
    h = jnp.dot(x_ref[...], w1_ref[...], preferred_element_type=jnp.float32)
    hp = jnp.concatenate([h, jnp.zeros((_ACC_ROWS - _N, _DIM), jnp.float32)], 0)
    h3 = hp.reshape(_VR, 8, _DIM)
    kk = lax.broadcasted_iota(jnp.int32, (_DIM, 128), 0)
    cc = lax.broadcasted_iota(jnp.int32, (_DIM, 128), 1)
    hv = jnp.zeros((_VR, 128), jnp.float32)
    for j in range(8):
        ej = jnp.where(cc == kk + _DIM * j, 1.0, 0.0)
        hv = hv + jnp.dot(h3[:, j, :], ej, preferred_element_type=jnp.float32)
    hv_ref[...] = hv


_tc1a = pl.pallas_call(
    _tc1a_body,
    out_shape=jax.ShapeDtypeStruct((_VR, 128), jnp.float32),
)


def _tc1b_body(degp_ref, hv_ref, dis16_ref, hs_ref, hself_ref):
    degp = degp_ref[0] + degp_ref[1]
    disp = lax.rsqrt(degp + 1.0)[:, 0:8]
    r8 = lax.broadcasted_iota(jnp.int32, (8, 128), 0)
    c8 = lax.broadcasted_iota(jnp.int32, (8, 128), 1) // _DIM
    expand = jnp.where(r8 == c8, 1.0, 0.0)
    dis16 = jnp.dot(disp, expand, preferred_element_type=jnp.float32)[0:_VR, :]
    hs = hv_ref[...] * dis16
    dis16_ref[...] = dis16
    hs_ref[...] = hs
    hself_ref[...] = hs * dis16


_tc1b = pl.pallas_call(
    _tc1b_body,
    out_shape=(
        jax.ShapeDtypeStruct((_VR, 128), jnp.float32),
        jax.ShapeDtypeStruct((_VR, 128), jnp.float32),
        jax.ShapeDtypeStruct((_VR, 128), jnp.float32),
    ),
)


def _tile8(mat):
    r = lax.broadcasted_iota(jnp.int32, (128, 128), 0)
    c = lax.broadcasted_iota(jnp.int32, (128, 128), 1)
    tiled = jnp.tile(mat, (8, 8))
    return jnp.where(r // _DIM == c // _DIM, tiled, 0.0)


def _tc2_body(acc_ref, dis16_ref, hself_ref, b1_ref, w2_ref, gs_ref, gself_ref):
    av = acc_ref[0:_VR, :] + acc_ref[_VR:2 * _VR, :]
    dis16 = dis16_ref[...]
    b1v = jnp.tile(b1_ref[...], (8,))
    z = av * dis16 + hself_ref[...] + b1v[None, :]
    h2 = jnp.maximum(z, 0.0)
    g = jnp.dot(h2, _tile8(w2_ref[...]), preferred_element_type=jnp.float32)
    gs = g * dis16
    gs_ref[...] = gs
    gself_ref[...] = gs * dis16


_tc2 = pl.pallas_call(
    _tc2_body,
    out_shape=(
        jax.ShapeDtypeStruct((_VR, 128), jnp.float32),
        jax.ShapeDtypeStruct((_VR, 128), jnp.float32),
    ),
)


def _tc3_body(acc_ref, dis16_ref, gself_ref, b2_ref, out_ref):
    av = acc_ref[0:_VR, :] + acc_ref[_VR:2 * _VR, :]
    b2v = jnp.tile(b2_ref[...], (8,))
    logitsv = av * dis16_ref[...] + gself_ref[...] + b2v[None, :]
    m = jnp.concatenate(
        [jnp.broadcast_to(
            jnp.max(logitsv[:, _DIM * j:_DIM * (j + 1)], axis=1, keepdims=True),
            (_VR, _DIM)) for j in range(8)], axis=1)
    ex = jnp.exp(logitsv - m)
    lse = jnp.concatenate(
        [jnp.broadcast_to(
            jnp.log(jnp.sum(ex[:, _DIM * j:_DIM * (j + 1)], axis=1, keepdims=True)),
            (_VR, _DIM)) for j in range(8)], axis=1) + m
    out_ref[...] = (logitsv - lse)[0:_N * _DIM // 128, :]


_tc3 = pl.pallas_call(
    _tc3_body,
    out_shape=jax.ShapeDtypeStruct((_N * _DIM // 128, 128), jnp.float32),
)


def kernel(x, edge_index, W1, b1, W2, b2):
    ei = edge_index.astype(jnp.int32).reshape(-1)

    sc_degree, sc_agg = _sc_kernels()
    hvr = _tc1a(x, W1)
    degp = sc_degree(ei)
    dis16, hsv, hselfv = _tc1b(degp, hvr)
    acc1 = sc_agg(hsv.reshape(_ACC_ROWS, _DIM), ei).reshape(2 * _VR, 128)
    gsv, gselfv = _tc2(acc1, dis16, hselfv, b1, W2)
    acc2 = sc_agg(gsv.reshape(_ACC_ROWS, _DIM), ei).reshape(2 * _VR, 128)
    predv = _tc3(acc2, dis16, gselfv, b2)
    return predv.reshape(_N, _DIM)


# speedup vs baseline: 1.2652x; 1.0846x over previous
import functools

import jax
import jax.numpy as jnp
from jax import lax
from jax.experimental import pallas as pl
from jax.experimental.pallas import tpu as pltpu
from jax.experimental.pallas import tpu_sc as plsc

_N = 10000
_E = 320000
_DIM = 16

_NW = 32
_EPT = _E // _NW
_NM = 5
_MB = _EPT // _NM
_RPT = 632
_ACC_ROWS = _RPT * 16
_DN = 1280

@functools.cache
def _sc_kernels():
    mesh = plsc.VectorSubcoreMesh(
        core_axis_name="c", subcore_axis_name="s", num_cores=2, num_subcores=16
    )

    @functools.partial(
        pl.kernel,
        out_type=jax.ShapeDtypeStruct((2, _DN, 16), jnp.float32),
        mesh=mesh,
        scratch_types=[
            pltpu.VMEM((_EPT + 128,), jnp.int32),
            pltpu.VMEM((_DN, 16), jnp.float32),
            pltpu.VMEM((_DN,), jnp.int32),
            pltpu.VMEM_SHARED((_DN, 16), jnp.float32),
        ],
        compiler_params=pltpu.CompilerParams(
            needs_layout_passes=False, use_tc_tiling_on_sc=False),
    )
    def sc_degree(e_hbm, out_hbm, didx, deg, idr, deg_s):
        c = lax.axis_index("c")
        s = lax.axis_index("s")
        wid = c * 16 + s
        zeros = jnp.zeros((16,), jnp.float32)

        def zbody(i, _):
            for u in range(8):
                deg[i * 8 + u, :] = zeros
            return 0

        lax.fori_loop(0, _DN // 8, zbody, 0)
        pltpu.sync_copy(deg.at[pl.ds(0, _DN // 16)], deg_s.at[pl.ds(s * (_DN // 16), _DN // 16)])
        beg = _E + wid * _EPT
        algn = pl.multiple_of((beg // 128) * 128, 128)
        off = beg - (beg // 128) * 128
        pltpu.sync_copy(e_hbm.at[pl.ds(algn, _EPT + 128)], didx)
        ones = jnp.ones((16,), jnp.float32)

        def body(i, _):
            for u in range(25):
                idx = didx[pl.ds(off + (i * 25 + u) * 16, 16)]
                plsc.addupdate_scatter(deg, [idx >> 3, idx & 7], ones)
            return 0

        lax.fori_loop(0, _EPT // 400, body, 0)
        iota = lax.iota(jnp.int32, 16)

        def ibody(i, _):
            idr[pl.ds(i * 16, 16)] = iota + i * 16
            return 0

        lax.fori_loop(0, _DN // 16, ibody, 0)
        plsc.subcore_barrier()
        pltpu.sync_copy(deg, deg_s.at[idr], add=True)
        plsc.subcore_barrier()
        pltpu.sync_copy(deg_s.at[pl.ds(s * (_DN // 16), _DN // 16)], deg.at[pl.ds(0, _DN // 16)])
        pltpu.sync_copy(deg.at[pl.ds(0, _DN // 16)], out_hbm.at[c, pl.ds(s * (_DN // 16), _DN // 16)])

    @functools.partial(
        pl.kernel,
        out_type=jax.ShapeDtypeStruct((2, _ACC_ROWS, _DIM), jnp.float32),
        mesh=mesh,
        scratch_types=[
            pltpu.VMEM((_EPT + 128,), jnp.int32),
            pltpu.VMEM((_EPT + 128,), jnp.int32),
            [pltpu.VMEM((_MB, _DIM), jnp.float32)] * 2,
            pltpu.VMEM((_RPT, _DIM), jnp.float32),
            pltpu.VMEM_SHARED((_ACC_ROWS, _DIM), jnp.float32),
            pltpu.VMEM_SHARED((_ACC_ROWS, _DIM), jnp.float32),
            [pltpu.SemaphoreType.DMA] * 4,
        ],
        compiler_params=pltpu.CompilerParams(use_tc_tiling_on_sc=False),
    )
    def sc_agg(tab_hbm, e_hbm, out_hbm, sidx, didx, rows, buf, acc, tabs, sems):
        c = lax.axis_index("c")
        s = lax.axis_index("s")
        wid = c * 16 + s
        gsem = [sems[0], sems[1]]
        ssem = [sems[2], sems[3]]
        zeros = jnp.zeros((16,), jnp.float32)

        def zbody(i, _):
            for u in range(8):
                buf[i * 8 + u, :] = zeros
            return 0

        sbeg = wid * _EPT
        dbeg = _E + wid * _EPT
        soff = sbeg - (sbeg // 128) * 128
        doff = dbeg - (dbeg // 128) * 128
        pltpu.make_async_copy(
            e_hbm.at[pl.ds(pl.multiple_of((sbeg // 128) * 128, 128), _EPT + 128)],
            sidx, gsem[0]).start()
        pltpu.make_async_copy(
            e_hbm.at[pl.ds(pl.multiple_of((dbeg // 128) * 128, 128), _EPT + 128)],
            didx, gsem[1]).start()
        pltpu.sync_copy(tab_hbm.at[pl.ds(s * _RPT, _RPT)], buf)
        pltpu.sync_copy(buf, tabs.at[pl.ds(s * _RPT, _RPT)])
        lax.fori_loop(0, _RPT // 8, zbody, 0)
        pltpu.sync_copy(buf, acc.at[pl.ds(s * _RPT, _RPT)])
        pltpu.make_async_copy(
            e_hbm.at[pl.ds(pl.multiple_of((sbeg // 128) * 128, 128), _EPT + 128)],
            sidx, gsem[0]).wait()
        pltpu.make_async_copy(
            e_hbm.at[pl.ds(pl.multiple_of((dbeg // 128) * 128, 128), _EPT + 128)],
            didx, gsem[1]).wait()
        plsc.subcore_barrier()

        def gat(m, b):
            return pltpu.make_async_copy(
                tabs.at[sidx.at[pl.ds(soff + m * _MB, _MB)]], rows[b], gsem[b])

        def sca(m, b):
            return pltpu.make_async_copy(
                rows[b], acc.at[didx.at[pl.ds(doff + m * _MB, _MB)]], ssem[b])

        gat(0, 0).start()
        for m in range(_NM):
            b = m % 2
            gat(m, b).wait()
            if m + 1 < _NM:
                if m >= 1:
                    sca(m - 1, 1 - b).wait()
                gat(m + 1, 1 - b).start()
            pltpu.async_copy(rows[b], acc.at[didx.at[pl.ds(doff + m * _MB, _MB)]],
                             ssem[b], add=True)
        sca(_NM - 2, (_NM - 2) % 2).wait()
        sca(_NM - 1, (_NM - 1) % 2).wait()
        plsc.subcore_barrier()
        pltpu.sync_copy(acc.at[pl.ds(s * _RPT, _RPT)], buf)
        pltpu.sync_copy(buf, out_hbm.at[c, pl.ds(s * _RPT, _RPT)])

    return sc_degree, sc_agg


_VR = _ACC_ROWS * _DIM // 128


def _tc1a_body(x_ref, w1_ref, hv_ref):
    h = jnp.dot(x_ref[...], w1_ref[...], preferred_element_type=jnp.float32)
    hp = jnp.concatenate([h, jnp.zeros((_ACC_ROWS - _N, _DIM), jnp.float32)], 0)
    h3 = hp.reshape(_VR, 8, _DIM)
    kk = lax.broadcasted_iota(jnp.int32, (_DIM, 128), 0)
    cc = lax.broadcasted_iota(jnp.int32, (_DIM, 128), 1)
    hv = jnp.zeros((_VR, 128), jnp.float32)
    for j in range(8):
        ej = jnp.where(cc == kk + _DIM * j, 1.0, 0.0)
        hv = hv + jnp.dot(h3[:, j, :], ej, preferred_element_type=jnp.float32)
    hv_ref[...] = hv


_tc1a = pl.pallas_call(
    _tc1a_body,
    out_shape=jax.ShapeDtypeStruct((_VR, 128), jnp.float32),
)


def _tc1b_body(degp_ref, hv_ref, dis16_ref, hs_ref, hself_ref):
    degp = degp_ref[0] + degp_ref[1]
    disp = lax.rsqrt(degp + 1.0)[:, 0:8]
    r8 = lax.broadcasted_iota(jnp.int32, (8, 128), 0)
    c8 = lax.broadcasted_iota(jnp.int32, (8, 128), 1) // _DIM
    expand = jnp.where(r8 == c8, 1.0, 0.0)
    dis16 = jnp.dot(disp, expand, preferred_element_type=jnp.float32)[0:_VR, :]
    hs = hv_ref[...] * dis16
    dis16_ref[...] = dis16
    hs_ref[...] = hs
    hself_ref[...] = hs * dis16


_tc1b = pl.pallas_call(
    _tc1b_body,
    out_shape=(
        jax.ShapeDtypeStruct((_VR, 128), jnp.float32),
        jax.ShapeDtypeStruct((_VR, 128), jnp.float32),
        jax.ShapeDtypeStruct((_VR, 128), jnp.float32),
    ),
)


def _tile8(mat):
    r = lax.broadcasted_iota(jnp.int32, (128, 128), 0)
    c = lax.broadcasted_iota(jnp.int32, (128, 128), 1)
    tiled = jnp.tile(mat, (8, 8))
    return jnp.where(r // _DIM == c // _DIM, tiled, 0.0)


def _tc2_body(acc_ref, dis16_ref, hself_ref, b1_ref, w2_ref, gs_ref, gself_ref):
    av = acc_ref[0:_VR, :] + acc_ref[_VR:2 * _VR, :]
    dis16 = dis16_ref[...]
    b1v = jnp.tile(b1_ref[...], (8,))
    z = av * dis16 + hself_ref[...] + b1v[None, :]
    h2 = jnp.maximum(z, 0.0)
    g = jnp.dot(h2, _tile8(w2_ref[...]), preferred_element_type=jnp.float32)
    gs = g * dis16
    gs_ref[...] = gs
    gself_ref[...] = gs * dis16


_tc2 = pl.pallas_call(
    _tc2_body,
    out_shape=(
        jax.ShapeDtypeStruct((_VR, 128), jnp.float32),
        jax.ShapeDtypeStruct((_VR, 128), jnp.float32),
    ),
)


def _tc3_body(acc_ref, dis16_ref, gself_ref, b2_ref, out_ref):
    av = acc_ref[0:_VR, :] + acc_ref[_VR:2 * _VR, :]
    b2v = jnp.tile(b2_ref[...], (8,))
    logitsv = av * dis16_ref[...] + gself_ref[...] + b2v[None, :]
    m = jnp.concatenate(
        [jnp.broadcast_to(
            jnp.max(logitsv[:, _DIM * j:_DIM * (j + 1)], axis=1, keepdims=True),
            (_VR, _DIM)) for j in range(8)], axis=1)
    ex = jnp.exp(logitsv - m)
    lse = jnp.concatenate(
        [jnp.broadcast_to(
            jnp.log(jnp.sum(ex[:, _DIM * j:_DIM * (j + 1)], axis=1, keepdims=True)),
            (_VR, _DIM)) for j in range(8)], axis=1) + m
    out_ref[...] = (logitsv - lse)[0:_N * _DIM // 128, :]


_tc3 = pl.pallas_call(
    _tc3_body,
    out_shape=jax.ShapeDtypeStruct((_N * _DIM // 128, 128), jnp.float32),
)


def kernel(x, edge_index, W1, b1, W2, b2):
    ei = edge_index.astype(jnp.int32).reshape(-1)

    sc_degree, sc_agg = _sc_kernels()
    hvr = _tc1a(x, W1)
    degp = sc_degree(ei)
    dis16, hsv, hselfv = _tc1b(degp, hvr)
    acc1 = sc_agg(hsv.reshape(_ACC_ROWS, _DIM), ei).reshape(2 * _VR, 128)
    gsv, gselfv = _tc2(acc1, dis16, hselfv, b1, W2)
    acc2 = sc_agg(gsv.reshape(_ACC_ROWS, _DIM), ei).reshape(2 * _VR, 128)
    predv = _tc3(acc2, dis16, gselfv, b2)
    return predv.reshape(_N, _DIM)


# speedup vs baseline: 1.2807x; 1.0122x over previous
import functools

import jax
import jax.numpy as jnp
from jax import lax
from jax.experimental import pallas as pl
from jax.experimental.pallas import tpu as pltpu
from jax.experimental.pallas import tpu_sc as plsc

_N = 10000
_E = 320000
_DIM = 16

_NW = 32
_EPT = _E // _NW
_NM = 5
_MB = _EPT // _NM
_RPT = 632
_ACC_ROWS = _RPT * 16
_DN = 1280

@functools.cache
def _sc_kernels():
    mesh = plsc.VectorSubcoreMesh(
        core_axis_name="c", subcore_axis_name="s", num_cores=2, num_subcores=16
    )

    @functools.partial(
        pl.kernel,
        out_type=jax.ShapeDtypeStruct((2, _DN, 16), jnp.float32),
        mesh=mesh,
        scratch_types=[
            pltpu.VMEM((_EPT + 128,), jnp.int32),
            pltpu.VMEM((_DN, 16), jnp.float32),
            pltpu.VMEM((_DN,), jnp.int32),
            pltpu.VMEM_SHARED((_DN, 16), jnp.float32),
            pltpu.SemaphoreType.DMA,
        ],
        compiler_params=pltpu.CompilerParams(
            needs_layout_passes=False, use_tc_tiling_on_sc=False),
    )
    def sc_degree(e_hbm, out_hbm, didx, deg, idr, deg_s, dsem):
        c = lax.axis_index("c")
        s = lax.axis_index("s")
        wid = c * 16 + s
        zeros = jnp.zeros((16,), jnp.float32)

        def zbody(i, _):
            for u in range(8):
                deg[i * 8 + u, :] = zeros
            return 0

        beg = _E + wid * _EPT
        algn = pl.multiple_of((beg // 128) * 128, 128)
        off = beg - (beg // 128) * 128
        pltpu.make_async_copy(e_hbm.at[pl.ds(algn, _EPT + 128)], didx, dsem).start()
        lax.fori_loop(0, _DN // 8, zbody, 0)
        pltpu.sync_copy(deg.at[pl.ds(0, _DN // 16)], deg_s.at[pl.ds(s * (_DN // 16), _DN // 16)])
        pltpu.make_async_copy(e_hbm.at[pl.ds(algn, _EPT + 128)], didx, dsem).wait()
        ones = jnp.ones((16,), jnp.float32)

        def body(i, _):
            for u in range(25):
                idx = didx[pl.ds(off + (i * 25 + u) * 16, 16)]
                plsc.addupdate_scatter(deg, [idx >> 3, idx & 7], ones)
            return 0

        lax.fori_loop(0, _EPT // 400, body, 0)
        iota = lax.iota(jnp.int32, 16)

        def ibody(i, _):
            idr[pl.ds(i * 16, 16)] = iota + i * 16
            return 0

        lax.fori_loop(0, _DN // 16, ibody, 0)
        plsc.subcore_barrier()
        pltpu.sync_copy(deg, deg_s.at[idr], add=True)
        plsc.subcore_barrier()
        pltpu.sync_copy(deg_s.at[pl.ds(s * (_DN // 16), _DN // 16)], deg.at[pl.ds(0, _DN // 16)])
        pltpu.sync_copy(deg.at[pl.ds(0, _DN // 16)], out_hbm.at[c, pl.ds(s * (_DN // 16), _DN // 16)])

    @functools.partial(
        pl.kernel,
        out_type=jax.ShapeDtypeStruct((2, _ACC_ROWS, _DIM), jnp.float32),
        mesh=mesh,
        scratch_types=[
            pltpu.VMEM((_EPT + 128,), jnp.int32),
            pltpu.VMEM((_EPT + 128,), jnp.int32),
            [pltpu.VMEM((_MB, _DIM), jnp.float32)] * 2,
            pltpu.VMEM((_RPT, _DIM), jnp.float32),
            pltpu.VMEM_SHARED((_ACC_ROWS, _DIM), jnp.float32),
            pltpu.VMEM_SHARED((_ACC_ROWS, _DIM), jnp.float32),
            [pltpu.SemaphoreType.DMA] * 4,
        ],
        compiler_params=pltpu.CompilerParams(use_tc_tiling_on_sc=False),
    )
    def sc_agg(tab_hbm, e_hbm, out_hbm, sidx, didx, rows, buf, acc, tabs, sems):
        c = lax.axis_index("c")
        s = lax.axis_index("s")
        wid = c * 16 + s
        gsem = [sems[0], sems[1]]
        ssem = [sems[2], sems[3]]
        zeros = jnp.zeros((16,), jnp.float32)

        def zbody(i, _):
            for u in range(8):
                buf[i * 8 + u, :] = zeros
            return 0

        sbeg = wid * _EPT
        dbeg = _E + wid * _EPT
        soff = sbeg - (sbeg // 128) * 128
        doff = dbeg - (dbeg // 128) * 128
        pltpu.make_async_copy(
            e_hbm.at[pl.ds(pl.multiple_of((sbeg // 128) * 128, 128), _EPT + 128)],
            sidx, gsem[0]).start()
        pltpu.make_async_copy(
            e_hbm.at[pl.ds(pl.multiple_of((dbeg // 128) * 128, 128), _EPT + 128)],
            didx, gsem[1]).start()
        pltpu.sync_copy(tab_hbm.at[pl.ds(s * _RPT, _RPT)], buf)
        pltpu.sync_copy(buf, tabs.at[pl.ds(s * _RPT, _RPT)])
        lax.fori_loop(0, _RPT // 8, zbody, 0)
        pltpu.sync_copy(buf, acc.at[pl.ds(s * _RPT, _RPT)])
        pltpu.make_async_copy(
            e_hbm.at[pl.ds(pl.multiple_of((sbeg // 128) * 128, 128), _EPT + 128)],
            sidx, gsem[0]).wait()
        pltpu.make_async_copy(
            e_hbm.at[pl.ds(pl.multiple_of((dbeg // 128) * 128, 128), _EPT + 128)],
            didx, gsem[1]).wait()
        plsc.subcore_barrier()

        def gat(m, b):
            return pltpu.make_async_copy(
                tabs.at[sidx.at[pl.ds(soff + m * _MB, _MB)]], rows[b], gsem[b])

        def sca(m, b):
            return pltpu.make_async_copy(
                rows[b], acc.at[didx.at[pl.ds(doff + m * _MB, _MB)]], ssem[b])

        gat(0, 0).start()
        for m in range(_NM):
            b = m % 2
            gat(m, b).wait()
            if m + 1 < _NM:
                if m >= 1:
                    sca(m - 1, 1 - b).wait()
                gat(m + 1, 1 - b).start()
            pltpu.async_copy(rows[b], acc.at[didx.at[pl.ds(doff + m * _MB, _MB)]],
                             ssem[b], add=True)
        sca(_NM - 2, (_NM - 2) % 2).wait()
        sca(_NM - 1, (_NM - 1) % 2).wait()
        plsc.subcore_barrier()
        pltpu.sync_copy(acc.at[pl.ds(s * _RPT, _RPT)], buf)
        pltpu.sync_copy(buf, out_hbm.at[c, pl.ds(s * _RPT, _RPT)])

    return sc_degree, sc_agg


_VR = _ACC_ROWS * _DIM // 128


def _tc1a_body(x_ref, w1_ref, hv_ref):
    h = jnp.dot(x_ref[...], w1_ref[...], preferred_element_type=jnp.float32)
    hp = jnp.concatenate([h, jnp.zeros((_ACC_ROWS - _N, _DIM), jnp.float32)], 0)
    h3 = hp.reshape(_VR, 8, _DIM)
    kk = lax.broadcasted_iota(jnp.int32, (_DIM, 128), 0)
    cc = lax.broadcasted_iota(jnp.int32, (_DIM, 128), 1)
    hv = jnp.zeros((_VR, 128), jnp.float32)
    for j in range(8):
        ej = jnp.where(cc == kk + _DIM * j, 1.0, 0.0)
        hv = hv + jnp.dot(h3[:, j, :], ej, preferred_element_type=jnp.float32)
    hv_ref[...] = hv


_tc1a = pl.pallas_call(
    _tc1a_body,
    out_shape=jax.ShapeDtypeStruct((_VR, 128), jnp.float32),
)


def _tc1b_body(degp_ref, hv_ref, dis16_ref, hs_ref, hself_ref):
    degp = degp_ref[0] + degp_ref[1]
    disp = lax.rsqrt(degp + 1.0)[:, 0:8]
    r8 = lax.broadcasted_iota(jnp.int32, (8, 128), 0)
    c8 = lax.broadcasted_iota(jnp.int32, (8, 128), 1) // _DIM
    expand = jnp.where(r8 == c8, 1.0, 0.0)
    dis16 = jnp.dot(disp, expand, preferred_element_type=jnp.float32)[0:_VR, :]
    hs = hv_ref[...] * dis16
    dis16_ref[...] = dis16
    hs_ref[...] = hs
    hself_ref[...] = hs * dis16


_tc1b = pl.pallas_call(
    _tc1b_body,
    out_shape=(
        jax.ShapeDtypeStruct((_VR, 128), jnp.float32),
        jax.ShapeDtypeStruct((_VR, 128), jnp.float32),
        jax.ShapeDtypeStruct((_VR, 128), jnp.float32),
    ),
)


def _tile8(mat):
    r = lax.broadcasted_iota(jnp.int32, (128, 128), 0)
    c = lax.broadcasted_iota(jnp.int32, (128, 128), 1)
    tiled = jnp.tile(mat, (8, 8))
    return jnp.where(r // _DIM == c // _DIM, tiled, 0.0)


def _tc2_body(acc_ref, dis16_ref, hself_ref, b1_ref, w2_ref, gs_ref, gself_ref):
    av = acc_ref[0:_VR, :] + acc_ref[_VR:2 * _VR, :]
    dis16 = dis16_ref[...]
    b1v = jnp.tile(b1_ref[...], (8,))
    z = av * dis16 + hself_ref[...] + b1v[None, :]
    h2 = jnp.maximum(z, 0.0)
    g = jnp.dot(h2, _tile8(w2_ref[...]), preferred_element_type=jnp.float32)
    gs = g * dis16
    gs_ref[...] = gs
    gself_ref[...] = gs * dis16


_tc2 = pl.pallas_call(
    _tc2_body,
    out_shape=(
        jax.ShapeDtypeStruct((_VR, 128), jnp.float32),
        jax.ShapeDtypeStruct((_VR, 128), jnp.float32),
    ),
)


def _tc3_body(acc_ref, dis16_ref, gself_ref, b2_ref, out_ref):
    av = acc_ref[0:_VR, :] + acc_ref[_VR:2 * _VR, :]
    b2v = jnp.tile(b2_ref[...], (8,))
    logitsv = av * dis16_ref[...] + gself_ref[...] + b2v[None, :]
    m = jnp.concatenate(
        [jnp.broadcast_to(
            jnp.max(logitsv[:, _DIM * j:_DIM * (j + 1)], axis=1, keepdims=True),
            (_VR, _DIM)) for j in range(8)], axis=1)
    ex = jnp.exp(logitsv - m)
    lse = jnp.concatenate(
        [jnp.broadcast_to(
            jnp.log(jnp.sum(ex[:, _DIM * j:_DIM * (j + 1)], axis=1, keepdims=True)),
            (_VR, _DIM)) for j in range(8)], axis=1) + m
    out_ref[...] = (logitsv - lse)[0:_N * _DIM // 128, :]


_tc3 = pl.pallas_call(
    _tc3_body,
    out_shape=jax.ShapeDtypeStruct((_N * _DIM // 128, 128), jnp.float32),
)


def kernel(x, edge_index, W1, b1, W2, b2):
    ei = edge_index.astype(jnp.int32).reshape(-1)

    sc_degree, sc_agg = _sc_kernels()
    hvr = _tc1a(x, W1)
    degp = sc_degree(ei)
    dis16, hsv, hselfv = _tc1b(degp, hvr)
    acc1 = sc_agg(hsv.reshape(_ACC_ROWS, _DIM), ei).reshape(2 * _VR, 128)
    gsv, gselfv = _tc2(acc1, dis16, hselfv, b1, W2)
    acc2 = sc_agg(gsv.reshape(_ACC_ROWS, _DIM), ei).reshape(2 * _VR, 128)
    predv = _tc3(acc2, dis16, gselfv, b2)
    return predv.reshape(_N, _DIM)
